# baseline (speedup 1.0000x reference)
import jax
import jax.numpy as jnp
from jax.experimental import pallas as pl
from jax.experimental.pallas import tpu as pltpu

_EPS = 1e-12
_BATCH = 8


def _vlad_one(x1_f32, x2_f32, wb1, wb2, c_ref):
    Ch = x1_f32.shape[0]
    xb1 = x1_f32.astype(jnp.bfloat16)
    xb2 = x2_f32.astype(jnp.bfloat16)

    ones_row = jnp.ones((1, Ch), jnp.bfloat16)
    nrm2 = (jnp.dot(ones_row, xb1 * xb1,
                    preferred_element_type=jnp.float32) +
            jnp.dot(ones_row, xb2 * xb2,
                    preferred_element_type=jnp.float32))
    nrm = jnp.maximum(jnp.sqrt(nrm2), _EPS)
    rinv = 1.0 / nrm

    raw = (jnp.dot(wb1, xb1, preferred_element_type=jnp.float32) +
           jnp.dot(wb2, xb2, preferred_element_type=jnp.float32))
    e = jnp.exp(raw * rinv)
    scale = rinv / jnp.sum(e, axis=0, keepdims=True)
    ab = (e * scale).astype(jnp.bfloat16)

    v1 = jax.lax.dot_general(
        ab, xb1, (((1,), (1,)), ((), ())),
        preferred_element_type=jnp.float32)
    v2 = jax.lax.dot_general(
        ab, xb2, (((1,), (1,)), ((), ())),
        preferred_element_type=jnp.float32)
    vlad = jnp.concatenate([v1, v2], axis=1)
    nrm8 = jnp.broadcast_to(nrm, (8, nrm.shape[1])).astype(jnp.bfloat16)
    asum = jax.lax.dot_general(
        ab, nrm8, (((1,), (1,)), ((), ())),
        preferred_element_type=jnp.float32)[:, 0:1]
    vlad = vlad - asum * c_ref[...]

    rn2 = jnp.sum(vlad * vlad, axis=1, keepdims=True)
    vlad = vlad / jnp.maximum(jnp.sqrt(rn2), _EPS)

    gn2 = jnp.sum(vlad * vlad, keepdims=True)
    return vlad / jnp.maximum(jnp.sqrt(gn2), _EPS)


def _vlad_body(x1_ref, x2_ref, w_ref, c_ref, o_ref):
    wb = w_ref[...].astype(jnp.bfloat16)
    Ch = wb.shape[1] // 2
    wb1 = wb[:, :Ch]
    wb2 = wb[:, Ch:]
    for i in range(_BATCH):
        o_ref[i] = _vlad_one(x1_ref[i], x2_ref[i], wb1, wb2, c_ref)


def kernel(x, conv_w, centroids):
    N, C, H, W = x.shape
    K = conv_w.shape[0]
    S = H * W
    Ch = C // 2
    xf = x.reshape(N, C, S)

    out = pl.pallas_call(
        _vlad_body,
        grid=(N // _BATCH,),
        in_specs=[
            pl.BlockSpec((_BATCH, Ch, S), lambda n: (n, 0, 0)),
            pl.BlockSpec((_BATCH, Ch, S), lambda n: (n, 1, 0)),
            pl.BlockSpec((K, C), lambda n: (0, 0)),
            pl.BlockSpec((K, C), lambda n: (0, 0)),
        ],
        out_specs=pl.BlockSpec((_BATCH, K, C), lambda n: (n, 0, 0)),
        out_shape=jax.ShapeDtypeStruct((N, K, C), jnp.float32),
        compiler_params=pltpu.CompilerParams(
            dimension_semantics=("parallel",),
            vmem_limit_bytes=50 * 1024 * 1024,
        ),
    )(xf, xf, conv_w, centroids)
    return out.reshape(N, K * C)
